# baseline (speedup 1.0000x reference)
import functools

import jax
import jax.numpy as jnp
import numpy as np
from jax import lax
from jax.experimental import pallas as pl
from jax.experimental.pallas import tpu as pltpu
from jax.experimental.pallas import tpu_sc as plsc

_D = 128
_B = 16
_L = 512
_LV = 9
_N1 = _B * _L // 2
_CH = 8
_CR = _N1 // _CH


def _bitrev_perm():
    l = np.arange(_L)
    rev = np.zeros(_L, dtype=np.int64)
    for b in range(_LV):
        rev |= ((l >> b) & 1) << (_LV - 1 - b)
    perm = np.zeros(_B * _L, dtype=np.int32)
    t = np.arange(_B)
    perm[rev[:, None] * _B + (_B - 1 - t)[None, :]] = (t[None, :] * _L + l[:, None])
    return perm


def _sc_gather(emb, tokens, perm):
    (B,) = tokens.shape
    V, D = emb.shape
    info = plsc.get_sparse_core_info()
    nw = info.num_cores * info.num_subcores
    bpw = B // nw
    mesh = plsc.VectorSubcoreMesh(core_axis_name="c", subcore_axis_name="s")

    @functools.partial(
        pl.kernel,
        mesh=mesh,
        out_type=jax.ShapeDtypeStruct((B, D), emb.dtype),
        scratch_types=[
            pltpu.VMEM((bpw,), jnp.int32),
            pltpu.VMEM((bpw,), jnp.int32),
            pltpu.VMEM((bpw, D), emb.dtype),
            pltpu.SemaphoreType.DMA,
        ],
    )
    def gather_k(perm_hbm, tok_hbm, table_hbm, out_hbm, perm_v, idx_v, rows_v,
                 sem):
        wid = lax.axis_index("s") * info.num_cores + lax.axis_index("c")
        base = wid * bpw
        pltpu.sync_copy(perm_hbm.at[pl.ds(base, bpw)], perm_v)
        pltpu.async_copy(tok_hbm.at[perm_v], idx_v, sem).wait()
        pltpu.async_copy(table_hbm.at[idx_v], rows_v, sem).wait()
        pltpu.sync_copy(rows_v, out_hbm.at[pl.ds(base, bpw)])

    return gather_k(perm, tokens, emb)


def _sigp(v):
    return 0.5 * jnp.tanh(v) + 0.5


def _binary_unary(h1, h2, m1, m2, wb_ref, bb_ref, wbs_ref, bbs_ref, wu_ref,
                  bu_ref, wus_ref, bus_ref):
    f32 = jnp.float32
    g = (jnp.dot(h1, wb_ref[0:_D, :], preferred_element_type=f32)
         + jnp.dot(h2, wb_ref[_D:2 * _D, :], preferred_element_type=f32)
         + bb_ref[...])
    i = _sigp(g[:, 0:_D])
    f1 = _sigp(g[:, _D:2 * _D])
    f2 = _sigp(g[:, 2 * _D:3 * _D])
    o = _sigp(g[:, 3 * _D:4 * _D])
    u = jnp.tanh(g[:, 4 * _D:5 * _D])
    c = i * u
    if m1:
        c = c + f1 * m1[0] + f2 * m2[0]
    hb = o * jnp.tanh(c)
    hbb = hb.astype(jnp.bfloat16)
    alpha = _sigp(
        jnp.dot(h1, wbs_ref[0:_D, :], preferred_element_type=f32)
        + jnp.dot(h2, wbs_ref[_D:2 * _D, :], preferred_element_type=f32)
        + bbs_ref[...])
    mb = [c]
    if m1:
        mb += [alpha * a + (1.0 - alpha) * b for a, b in zip(m1[:3], m2[:3])]
    gu = (jnp.dot(hbb, wu_ref[...], preferred_element_type=f32) + bu_ref[...])
    iu = _sigp(gu[:, 0:_D])
    fu = _sigp(gu[:, _D:2 * _D])
    ou = _sigp(gu[:, 2 * _D:3 * _D])
    uu = jnp.tanh(gu[:, 3 * _D:4 * _D])
    cu = iu * uu + fu * mb[0]
    hu = ou * jnp.tanh(cu)
    beta = _sigp(
        jnp.dot(hbb, wus_ref[...], preferred_element_type=f32) + bus_ref[...])
    return hu, [cu] + [beta * s for s in mb[:3]]


def _tree_body(h0a_ref, h0b_ref, wb_ref, bb_ref, wbs_ref, bbs_ref, wu_ref,
               bu_ref, wus_ref, bus_ref, wo_ref, bo_ref, out_ref,
               hs_ref, ms0_ref, ms1_ref):
    j = pl.program_id(0)
    bf16 = jnp.bfloat16

    @pl.when(j < _CH)
    def _level1():
        h1 = h0a_ref[...].astype(bf16)
        h2 = h0b_ref[...].astype(bf16)
        hu, mu = _binary_unary(h1, h2, [], [], wb_ref, bb_ref, wbs_ref,
                               bbs_ref, wu_ref, bu_ref, wus_ref, bus_ref)
        base = j * _CR
        hs_ref[pl.ds(base, _CR), :] = hu
        ms0_ref[pl.ds(base, _CR), :] = mu[0]
        ms1_ref[pl.ds(base, _CR), :] = mu[1]

    @pl.when(j == _CH)
    def _rest():
        h = hs_ref[...]
        m = [ms0_ref[...], ms1_ref[...]]
        rows = _N1
        while rows > _B:
            half = rows // 2
            h1 = h[:half].astype(bf16)
            h2 = h[half:].astype(bf16)
            m1 = [s[:half] for s in m]
            m2 = [s[half:] for s in m]
            h, m = _binary_unary(h1, h2, m1, m2, wb_ref, bb_ref, wbs_ref,
                                 bbs_ref, wu_ref, bu_ref, wus_ref, bus_ref)
            rows = half
        out_ref[...] = (
            jnp.dot(h.astype(bf16), wo_ref[...],
                    preferred_element_type=jnp.float32) + bo_ref[...])


def kernel(tokens, lengths, emb, Wb, bb, Wbs, bbs, Wu, bu, Wus, bus, Wo, bo):
    del lengths
    perm = jnp.asarray(_bitrev_perm())
    h0 = _sc_gather(emb, tokens.astype(jnp.int32), perm)

    bf16 = jnp.bfloat16
    f32 = jnp.float32
    Wb_s = jnp.concatenate([Wb[:, :4 * _D] * 0.5, Wb[:, 4 * _D:]],
                           axis=1).astype(bf16)
    bb_s = jnp.concatenate([bb[:4 * _D] * 0.5, bb[4 * _D:]]).reshape(1, -1)
    Wu_s = jnp.concatenate([Wu[:, :3 * _D] * 0.5, Wu[:, 3 * _D:]],
                           axis=1).astype(bf16)
    bu_s = jnp.concatenate([bu[:3 * _D] * 0.5, bu[3 * _D:]]).reshape(1, -1)
    Wbs_s = (Wbs * 0.5).astype(bf16)
    bbs_s = (bbs * 0.5).reshape(1, -1)
    Wus_s = (Wus * 0.5).astype(bf16)
    bus_s = (bus * 0.5).reshape(1, -1)

    def l1a(j):
        return (jnp.minimum(j, _CH - 1), 0)

    def l1b(j):
        return (_CH + jnp.minimum(j, _CH - 1), 0)

    def const(j):
        return (0, 0)

    grid = (_CH + 1,)
    in_specs = [
        pl.BlockSpec((_CR, _D), l1a),
        pl.BlockSpec((_CR, _D), l1b),
        pl.BlockSpec(Wb_s.shape, const),
        pl.BlockSpec(bb_s.shape, const),
        pl.BlockSpec(Wbs_s.shape, const),
        pl.BlockSpec(bbs_s.shape, const),
        pl.BlockSpec(Wu_s.shape, const),
        pl.BlockSpec(bu_s.shape, const),
        pl.BlockSpec(Wus_s.shape, const),
        pl.BlockSpec(bus_s.shape, const),
        pl.BlockSpec((_D, Wo.shape[1]), const),
        pl.BlockSpec((1, Wo.shape[1]), const),
    ]
    logits = pl.pallas_call(
        _tree_body,
        grid=grid,
        in_specs=in_specs,
        out_specs=pl.BlockSpec((_B, Wo.shape[1]), const),
        out_shape=jax.ShapeDtypeStruct((_B, Wo.shape[1]), jnp.float32),
        scratch_shapes=[
            pltpu.VMEM((_N1, _D), f32),
            pltpu.VMEM((_N1, _D), f32),
            pltpu.VMEM((_N1, _D), f32),
        ],
    )(h0, h0, Wb_s, bb_s, Wbs_s, bbs_s, Wu_s, bu_s, Wus_s, bus_s,
      Wo.astype(bf16), bo.reshape(1, -1))
    return logits


# speedup vs baseline: 1.0495x; 1.0495x over previous
import functools

import jax
import jax.numpy as jnp
import numpy as np
from jax import lax
from jax.experimental import pallas as pl
from jax.experimental.pallas import tpu as pltpu
from jax.experimental.pallas import tpu_sc as plsc

_D = 128
_B = 16
_L = 512
_LV = 9
_N1 = _B * _L // 2
_CH = 2
_CR = _N1 // _CH


def _bitrev_perm():
    l = np.arange(_L)
    rev = np.zeros(_L, dtype=np.int64)
    for b in range(_LV):
        rev |= ((l >> b) & 1) << (_LV - 1 - b)
    perm = np.zeros(_B * _L, dtype=np.int32)
    t = np.arange(_B)
    perm[rev[:, None] * _B + (_B - 1 - t)[None, :]] = (t[None, :] * _L + l[:, None])
    return perm


def _sc_gather(emb, tokens, perm):
    (B,) = tokens.shape
    V, D = emb.shape
    info = plsc.get_sparse_core_info()
    nw = info.num_cores * info.num_subcores
    bpw = B // nw
    mesh = plsc.VectorSubcoreMesh(core_axis_name="c", subcore_axis_name="s")

    @functools.partial(
        pl.kernel,
        mesh=mesh,
        out_type=jax.ShapeDtypeStruct((B, D), emb.dtype),
        scratch_types=[
            pltpu.VMEM((bpw,), jnp.int32),
            pltpu.VMEM((bpw,), jnp.int32),
            pltpu.VMEM((bpw, D), emb.dtype),
            pltpu.SemaphoreType.DMA,
        ],
    )
    def gather_k(perm_hbm, tok_hbm, table_hbm, out_hbm, perm_v, idx_v, rows_v,
                 sem):
        wid = lax.axis_index("s") * info.num_cores + lax.axis_index("c")
        base = wid * bpw
        pltpu.sync_copy(perm_hbm.at[pl.ds(base, bpw)], perm_v)
        pltpu.async_copy(tok_hbm.at[perm_v], idx_v, sem).wait()
        pltpu.async_copy(table_hbm.at[idx_v], rows_v, sem).wait()
        pltpu.sync_copy(rows_v, out_hbm.at[pl.ds(base, bpw)])

    return gather_k(perm, tokens, emb)


def _sigp(v):
    return 0.5 * jnp.tanh(v) + 0.5


def _binary_unary(h1, h2, m1, m2, wb_ref, bb_ref, wbs_ref, bbs_ref, wu_ref,
                  bu_ref, wus_ref, bus_ref):
    f32 = jnp.float32
    g = (jnp.dot(h1, wb_ref[0:_D, :], preferred_element_type=f32)
         + jnp.dot(h2, wb_ref[_D:2 * _D, :], preferred_element_type=f32)
         + bb_ref[...])
    i = _sigp(g[:, 0:_D])
    f1 = _sigp(g[:, _D:2 * _D])
    f2 = _sigp(g[:, 2 * _D:3 * _D])
    o = _sigp(g[:, 3 * _D:4 * _D])
    u = jnp.tanh(g[:, 4 * _D:5 * _D])
    c = i * u
    if m1:
        c = c + f1 * m1[0] + f2 * m2[0]
    hb = o * jnp.tanh(c)
    hbb = hb.astype(jnp.bfloat16)
    alpha = _sigp(
        jnp.dot(h1, wbs_ref[0:_D, :], preferred_element_type=f32)
        + jnp.dot(h2, wbs_ref[_D:2 * _D, :], preferred_element_type=f32)
        + bbs_ref[...])
    mb = [c]
    if m1:
        mb += [alpha * a + (1.0 - alpha) * b for a, b in zip(m1[:3], m2[:3])]
    gu = (jnp.dot(hbb, wu_ref[...], preferred_element_type=f32) + bu_ref[...])
    iu = _sigp(gu[:, 0:_D])
    fu = _sigp(gu[:, _D:2 * _D])
    ou = _sigp(gu[:, 2 * _D:3 * _D])
    uu = jnp.tanh(gu[:, 3 * _D:4 * _D])
    cu = iu * uu + fu * mb[0]
    hu = ou * jnp.tanh(cu)
    beta = _sigp(
        jnp.dot(hbb, wus_ref[...], preferred_element_type=f32) + bus_ref[...])
    return hu, [cu] + [beta * s for s in mb[:3]]


def _tree_body(h0a_ref, h0b_ref, wb_ref, bb_ref, wbs_ref, bbs_ref, wu_ref,
               bu_ref, wus_ref, bus_ref, wo_ref, bo_ref, out_ref,
               hs_ref, ms0_ref, ms1_ref):
    j = pl.program_id(0)
    bf16 = jnp.bfloat16

    @pl.when(j < _CH)
    def _level1():
        h1 = h0a_ref[...].astype(bf16)
        h2 = h0b_ref[...].astype(bf16)
        hu, mu = _binary_unary(h1, h2, [], [], wb_ref, bb_ref, wbs_ref,
                               bbs_ref, wu_ref, bu_ref, wus_ref, bus_ref)
        base = j * _CR
        hs_ref[pl.ds(base, _CR), :] = hu
        ms0_ref[pl.ds(base, _CR), :] = mu[0]
        ms1_ref[pl.ds(base, _CR), :] = mu[1]

    @pl.when(j == _CH)
    def _rest():
        h = hs_ref[...]
        m = [ms0_ref[...], ms1_ref[...]]
        rows = _N1
        while rows > _B:
            half = rows // 2
            h1 = h[:half].astype(bf16)
            h2 = h[half:].astype(bf16)
            m1 = [s[:half] for s in m]
            m2 = [s[half:] for s in m]
            h, m = _binary_unary(h1, h2, m1, m2, wb_ref, bb_ref, wbs_ref,
                                 bbs_ref, wu_ref, bu_ref, wus_ref, bus_ref)
            rows = half
        out_ref[...] = (
            jnp.dot(h.astype(bf16), wo_ref[...],
                    preferred_element_type=jnp.float32) + bo_ref[...])


def kernel(tokens, lengths, emb, Wb, bb, Wbs, bbs, Wu, bu, Wus, bus, Wo, bo):
    del lengths
    perm = jnp.asarray(_bitrev_perm())
    h0 = _sc_gather(emb, tokens.astype(jnp.int32), perm)

    bf16 = jnp.bfloat16
    f32 = jnp.float32
    Wb_s = jnp.concatenate([Wb[:, :4 * _D] * 0.5, Wb[:, 4 * _D:]],
                           axis=1).astype(bf16)
    bb_s = jnp.concatenate([bb[:4 * _D] * 0.5, bb[4 * _D:]]).reshape(1, -1)
    Wu_s = jnp.concatenate([Wu[:, :3 * _D] * 0.5, Wu[:, 3 * _D:]],
                           axis=1).astype(bf16)
    bu_s = jnp.concatenate([bu[:3 * _D] * 0.5, bu[3 * _D:]]).reshape(1, -1)
    Wbs_s = (Wbs * 0.5).astype(bf16)
    bbs_s = (bbs * 0.5).reshape(1, -1)
    Wus_s = (Wus * 0.5).astype(bf16)
    bus_s = (bus * 0.5).reshape(1, -1)

    def l1a(j):
        return (jnp.minimum(j, _CH - 1), 0)

    def l1b(j):
        return (_CH + jnp.minimum(j, _CH - 1), 0)

    def const(j):
        return (0, 0)

    grid = (_CH + 1,)
    in_specs = [
        pl.BlockSpec((_CR, _D), l1a),
        pl.BlockSpec((_CR, _D), l1b),
        pl.BlockSpec(Wb_s.shape, const),
        pl.BlockSpec(bb_s.shape, const),
        pl.BlockSpec(Wbs_s.shape, const),
        pl.BlockSpec(bbs_s.shape, const),
        pl.BlockSpec(Wu_s.shape, const),
        pl.BlockSpec(bu_s.shape, const),
        pl.BlockSpec(Wus_s.shape, const),
        pl.BlockSpec(bus_s.shape, const),
        pl.BlockSpec((_D, Wo.shape[1]), const),
        pl.BlockSpec((1, Wo.shape[1]), const),
    ]
    logits = pl.pallas_call(
        _tree_body,
        grid=grid,
        in_specs=in_specs,
        out_specs=pl.BlockSpec((_B, Wo.shape[1]), const),
        out_shape=jax.ShapeDtypeStruct((_B, Wo.shape[1]), jnp.float32),
        scratch_shapes=[
            pltpu.VMEM((_N1, _D), f32),
            pltpu.VMEM((_N1, _D), f32),
            pltpu.VMEM((_N1, _D), f32),
        ],
    )(h0, h0, Wb_s, bb_s, Wbs_s, bbs_s, Wu_s, bu_s, Wus_s, bus_s,
      Wo.astype(bf16), bo.reshape(1, -1))
    return logits
